# baseline (speedup 1.0000x reference)
import jax
import jax.numpy as jnp
from jax import lax
from jax.experimental import pallas as pl
from jax.experimental.pallas import tpu as pltpu
from jax.experimental.pallas import tpu_sc as plsc

EMB = 16
BATCH = 16384
NW = 32
BW = BATCH // NW
CHUNK = 128
NCHUNK = BW // CHUNK
REG = 1e-05


def _body(user_hbm, item_hbm, users_hbm, pos_hbm, neg_hbm, out_hbm,
          idx_u, idx_p, idx_n, rows_u, rows_p, rows_n, vout, sem):
    c = lax.axis_index("c")
    s = lax.axis_index("s")
    wid = s * 2 + c

    pltpu.sync_copy(users_hbm.at[pl.ds(wid * NCHUNK, NCHUNK)], idx_u)
    pltpu.sync_copy(pos_hbm.at[pl.ds(wid * NCHUNK, NCHUNK)], idx_p)
    pltpu.sync_copy(neg_hbm.at[pl.ds(wid * NCHUNK, NCHUNK)], idx_n)

    handles = []
    for j in range(NCHUNK):
        dst = pl.ds(j * CHUNK, CHUNK)
        handles.append(pltpu.async_copy(user_hbm.at[idx_u.at[j]], rows_u.at[dst], sem))
        handles.append(pltpu.async_copy(item_hbm.at[idx_p.at[j]], rows_p.at[dst], sem))
        handles.append(pltpu.async_copy(item_hbm.at[idx_n.at[j]], rows_n.at[dst], sem))
    for h in handles:
        h.wait()

    lane = lax.iota(jnp.int32, 16)
    perms = {h: lane ^ h for h in (1, 2, 4, 8)}
    masks = {h: (lane & h) == 0 for h in (1, 2, 4, 8)}

    def _take(v, ix):
        return v.at[ix].get(mode="promise_in_bounds")

    def block_step(j, carry):
        mf, sq = carry
        base = j * 16
        vs = []
        for k in range(16):
            u = rows_u[base + k, :]
            p = rows_p[base + k, :]
            n = rows_n[base + k, :]
            vs.append(u * (p - n))
            sq = sq + (u * u + (p * p + n * n))
        for h in (1, 2, 4, 8):
            vs = [jnp.where(masks[h], a + _take(a, perms[h]), b + _take(b, perms[h]))
                  for a, b in zip(vs[0::2], vs[1::2])]
        x = vs[0]
        y = jnp.exp(-jnp.abs(x))
        t = y / (y + 2.0)
        t2 = t * t
        poly = 1.0 + t2 * (1.0 / 3.0 + t2 * (1.0 / 5.0 + t2 * (1.0 / 7.0 + t2 * (1.0 / 9.0 + t2 * (1.0 / 11.0)))))
        mf = mf + (jnp.maximum(-x, 0.0) + 2.0 * t * poly)
        return (mf, sq)

    zero = jnp.zeros((16,), jnp.float32)
    mf_acc, sq_acc = lax.fori_loop(0, BW // 16, block_step, (zero, zero))

    def _allsum(v):
        for h in (8, 4, 2, 1):
            v = v + _take(v, perms[h])
        return v

    vec = jnp.where(lane == 0, _allsum(mf_acc),
                    jnp.where(lane == 1, _allsum(sq_acc), 0.0))
    vout[...] = vec
    pltpu.sync_copy(vout, out_hbm.at[wid])


def kernel(user_emb, item_emb, users, pos_items, neg_items):
    mesh = plsc.VectorSubcoreMesh(core_axis_name="c", subcore_axis_name="s")
    part = pl.kernel(
        _body,
        mesh=mesh,
        compiler_params=pltpu.CompilerParams(use_tc_tiling_on_sc=False),
        out_type=jax.ShapeDtypeStruct((NW, EMB), jnp.float32),
        scratch_types=[
            pltpu.VMEM((NCHUNK, CHUNK), jnp.int32),
            pltpu.VMEM((NCHUNK, CHUNK), jnp.int32),
            pltpu.VMEM((NCHUNK, CHUNK), jnp.int32),
            pltpu.VMEM((BW, EMB), jnp.float32),
            pltpu.VMEM((BW, EMB), jnp.float32),
            pltpu.VMEM((BW, EMB), jnp.float32),
            pltpu.VMEM((EMB,), jnp.float32),
            pltpu.SemaphoreType.DMA,
        ],
    )(
        user_emb,
        item_emb,
        users.astype(jnp.int32).reshape(BATCH // CHUNK, CHUNK),
        pos_items.astype(jnp.int32).reshape(BATCH // CHUNK, CHUNK),
        neg_items.astype(jnp.int32).reshape(BATCH // CHUNK, CHUNK),
    )
    mf_loss = jnp.sum(part[:, 0]) / BATCH
    emb_loss = REG * jnp.sum(part[:, 1])
    return (mf_loss, emb_loss)
